# baseline (speedup 1.0000x reference)
import functools

import jax
import jax.numpy as jnp
from jax import lax
from jax.experimental import pallas as pl
from jax.experimental.pallas import tpu as pltpu
from jax.experimental.pallas import tpu_sc as plsc

D = 128
NSLOT = 11
NC, NS = 2, 16
NW = NC * NS
R = 64
CH = 5
BLK = 512


def _sc_gather_sum(features, idxT, b_pad):
    bpw = b_pad // NW
    nchunks = bpw // R
    mesh = plsc.VectorSubcoreMesh(core_axis_name="c", subcore_axis_name="s")

    @functools.partial(
        pl.kernel,
        out_type=(jax.ShapeDtypeStruct((b_pad, D), jnp.float32),
                  jax.ShapeDtypeStruct((b_pad, D), jnp.float32)),
        mesh=mesh,
        scratch_types=[
            pltpu.VMEM((NSLOT, bpw), jnp.int32),
            pltpu.VMEM((CH, R, D), jnp.float32),
            pltpu.VMEM((CH, R, D), jnp.float32),
            pltpu.SemaphoreType.DMA,
            pltpu.SemaphoreType.DMA,
            [pltpu.SemaphoreType.DMA] * CH,
        ],
        compiler_params=pltpu.CompilerParams(use_tc_tiling_on_sc=False),
    )
    def k(feat_hbm, idxT_hbm, self_hbm, nsum_hbm, idx_v, sbuf, nbuf, ssem,
          osem, csems):
        wid = lax.axis_index("s") * NC + lax.axis_index("c")
        base = wid * bpw
        pltpu.sync_copy(idxT_hbm.at[wid], idx_v)

        def rnd(gi, carry):
            off0 = gi * (CH * R)
            selfs = [
                pltpu.async_copy(
                    feat_hbm.at[idx_v.at[0, pl.ds(off0 + k_ * R, R)]],
                    sbuf.at[k_], ssem)
                for k_ in range(CH)
            ]
            inits = [
                pltpu.async_copy(
                    feat_hbm.at[idx_v.at[1, pl.ds(off0 + k_ * R, R)]],
                    nbuf.at[k_], csems[k_])
                for k_ in range(CH)
            ]
            prev = inits
            for j in range(2, NSLOT):
                nxt = []
                for k_ in range(CH):
                    prev[k_].wait()
                    nxt.append(pltpu.async_copy(
                        feat_hbm.at[idx_v.at[j, pl.ds(off0 + k_ * R, R)]],
                        nbuf.at[k_], csems[k_], add=True))
                prev = nxt
            outs = []
            for k_ in range(CH):
                selfs[k_].wait()
                outs.append(pltpu.async_copy(
                    sbuf.at[k_], self_hbm.at[pl.ds(base + off0 + k_ * R, R)],
                    osem))
            for k_ in range(CH):
                prev[k_].wait()
                outs.append(pltpu.async_copy(
                    nbuf.at[k_], nsum_hbm.at[pl.ds(base + off0 + k_ * R, R)],
                    osem))
            for cp in outs:
                cp.wait()
            return carry

        lax.fori_loop(0, nchunks // CH, rnd, 0)

    return k(features, idxT)


def _tc_combine(self_rows, nsum, w1, w2):
    b_pad = self_rows.shape[0]

    def body(x1, x2, w1r, w2r, o):
        acc = jnp.dot(x1[...], w1r[...], preferred_element_type=jnp.float32)
        acc = acc + jnp.dot(x2[...] * jnp.float32(0.1), w2r[...],
                            preferred_element_type=jnp.float32)
        o[...] = jnp.maximum(acc, 0.0)

    return pl.pallas_call(
        body,
        grid=(b_pad // BLK,),
        in_specs=[
            pl.BlockSpec((BLK, D), lambda i: (i, 0)),
            pl.BlockSpec((BLK, D), lambda i: (i, 0)),
            pl.BlockSpec((D, D), lambda i: (0, 0)),
            pl.BlockSpec((D, D), lambda i: (0, 0)),
        ],
        out_specs=pl.BlockSpec((BLK, D), lambda i: (i, 0)),
        out_shape=jax.ShapeDtypeStruct((b_pad, D), jnp.float32),
    )(self_rows, nsum, w1, w2)


def kernel(features, weight, nodes, neigh_idx):
    b = nodes.shape[0]
    step = NW * R * CH
    b_pad = ((b + step - 1) // step) * step

    idx_all = jnp.concatenate(
        [nodes[:, None].astype(jnp.int32), neigh_idx.astype(jnp.int32)],
        axis=1).T
    idxT = jnp.pad(idx_all, ((0, 0), (0, b_pad - b)))
    idxT = idxT.reshape(NSLOT, NW, b_pad // NW).transpose(1, 0, 2)

    self_rows, nsum = _sc_gather_sum(features, idxT, b_pad)
    out = _tc_combine(self_rows, nsum, weight[:D], weight[D:])
    return out[:b]


# speedup vs baseline: 1.8582x; 1.8582x over previous
import functools

import jax
import jax.numpy as jnp
from jax import lax
from jax.experimental import pallas as pl
from jax.experimental.pallas import tpu as pltpu
from jax.experimental.pallas import tpu_sc as plsc

D = 128
NSLOT = 11
NC, NS = 2, 16
NW = NC * NS
R = 112
BLK = 512


def _sc_gather_sum(table, idxT, b_pad):
    bpw = b_pad // NW
    nchunks = bpw // R
    dt = table.dtype
    mesh = plsc.VectorSubcoreMesh(core_axis_name="c", subcore_axis_name="s")

    @functools.partial(
        pl.kernel,
        out_type=(jax.ShapeDtypeStruct((b_pad, D), dt),
                  jax.ShapeDtypeStruct((b_pad, D), dt)),
        mesh=mesh,
        scratch_types=[
            pltpu.VMEM((NSLOT, bpw), jnp.int32),
            pltpu.VMEM((2, R, D), dt),
            pltpu.VMEM((2, R, D), dt),
            pltpu.SemaphoreType.DMA,
            pltpu.SemaphoreType.DMA,
            pltpu.SemaphoreType.DMA,
        ],
        compiler_params=pltpu.CompilerParams(use_tc_tiling_on_sc=False),
    )
    def k(feat_hbm, idxT_hbm, self_hbm, nsum_hbm, idx_v, sbuf, nbuf, ssem,
          nsema, nsemb):
        wid = lax.axis_index("s") * NC + lax.axis_index("c")
        base = wid * bpw
        pltpu.sync_copy(idxT_hbm.at[wid], idx_v)

        def pair(pi, carry):
            off0 = (2 * pi) * R
            off1 = off0 + R
            s0 = pltpu.async_copy(
                feat_hbm.at[idx_v.at[0, pl.ds(off0, R)]], sbuf.at[0], ssem)
            s1 = pltpu.async_copy(
                feat_hbm.at[idx_v.at[0, pl.ds(off1, R)]], sbuf.at[1], ssem)
            a = pltpu.async_copy(
                feat_hbm.at[idx_v.at[1, pl.ds(off0, R)]], nbuf.at[0], nsema)
            bcp = pltpu.async_copy(
                feat_hbm.at[idx_v.at[1, pl.ds(off1, R)]], nbuf.at[1], nsemb)
            a.wait()
            bcp.wait()
            for j in range(2, NSLOT):
                aj = pltpu.async_copy(
                    feat_hbm.at[idx_v.at[j, pl.ds(off0, R)]], nbuf.at[0],
                    nsema, add=True)
                bj = pltpu.async_copy(
                    feat_hbm.at[idx_v.at[j, pl.ds(off1, R)]], nbuf.at[1],
                    nsemb, add=True)
                aj.wait()
                bj.wait()
            s0.wait()
            s1.wait()
            pltpu.sync_copy(sbuf.at[0], self_hbm.at[pl.ds(base + off0, R)])
            pltpu.sync_copy(sbuf.at[1], self_hbm.at[pl.ds(base + off1, R)])
            pltpu.sync_copy(nbuf.at[0], nsum_hbm.at[pl.ds(base + off0, R)])
            pltpu.sync_copy(nbuf.at[1], nsum_hbm.at[pl.ds(base + off1, R)])
            return carry

        lax.fori_loop(0, nchunks // 2, pair, 0)

    return k(table, idxT)


def _tc_combine(self_rows, nsum, w1, w2, b):
    def body(x1, x2, w1r, w2r, o):
        acc = jnp.dot(x1[...], w1r[...], preferred_element_type=jnp.float32)
        acc = acc + jnp.dot(x2[...], w2r[...],
                            preferred_element_type=jnp.float32) * \
            jnp.float32(0.1)
        o[...] = jnp.maximum(acc, 0.0)

    return pl.pallas_call(
        body,
        grid=((b + BLK - 1) // BLK,),
        in_specs=[
            pl.BlockSpec((BLK, D), lambda i: (i, 0)),
            pl.BlockSpec((BLK, D), lambda i: (i, 0)),
            pl.BlockSpec((D, D), lambda i: (0, 0)),
            pl.BlockSpec((D, D), lambda i: (0, 0)),
        ],
        out_specs=pl.BlockSpec((BLK, D), lambda i: (i, 0)),
        out_shape=jax.ShapeDtypeStruct((b, D), jnp.float32),
    )(self_rows, nsum, w1, w2)


def kernel(features, weight, nodes, neigh_idx):
    b = nodes.shape[0]
    step = NW * R * 2
    b_pad = ((b + step - 1) // step) * step

    table = features.astype(jnp.bfloat16)
    idx_all = jnp.concatenate(
        [nodes[:, None].astype(jnp.int32), neigh_idx.astype(jnp.int32)],
        axis=1).T
    idxT = jnp.pad(idx_all, ((0, 0), (0, b_pad - b)))
    idxT = idxT.reshape(NSLOT, NW, b_pad // NW).transpose(1, 0, 2)

    self_rows, nsum = _sc_gather_sum(table, idxT, b_pad)
    return _tc_combine(self_rows, nsum,
                       weight[:D].astype(jnp.bfloat16),
                       weight[D:].astype(jnp.bfloat16), b)


# speedup vs baseline: 2.5341x; 1.3638x over previous
import functools

import jax
import jax.numpy as jnp
from jax import lax
from jax.experimental import pallas as pl
from jax.experimental.pallas import tpu as pltpu
from jax.experimental.pallas import tpu_sc as plsc

D = 128
NSLOT = 11
NC, NS = 2, 16
NW = NC * NS
R = 112
BLK = 512


def _sc_gather_sum(table, idxT, b_pad):
    bpw = b_pad // NW
    nchunks = bpw // R
    dt = table.dtype
    mesh = plsc.VectorSubcoreMesh(core_axis_name="c", subcore_axis_name="s")

    @functools.partial(
        pl.kernel,
        out_type=(jax.ShapeDtypeStruct((b_pad, D), dt),
                  jax.ShapeDtypeStruct((b_pad, D), dt),
                  jax.ShapeDtypeStruct((b_pad, D), dt)),
        mesh=mesh,
        scratch_types=[
            pltpu.VMEM((NSLOT, bpw), jnp.int32),
            pltpu.VMEM((2, R, D), dt),
            pltpu.VMEM((4, R, D), dt),
            pltpu.SemaphoreType.DMA,
            pltpu.SemaphoreType.DMA,
            [pltpu.SemaphoreType.DMA] * 4,
        ],
        compiler_params=pltpu.CompilerParams(use_tc_tiling_on_sc=False),
    )
    def k(feat_hbm, idxT_hbm, self_hbm, nsa_hbm, nsb_hbm, idx_v, sbuf, nbuf,
          ssem, osem, csems):
        wid = lax.axis_index("s") * NC + lax.axis_index("c")
        base = wid * bpw
        pltpu.sync_copy(idxT_hbm.at[wid], idx_v)

        def chain_idx(q, off0, j):
            off = off0 + (q // 2) * R
            slot = 1 + (q % 2) * 5 + j
            return idx_v.at[slot, pl.ds(off, R)]

        def pair(pi, carry):
            off0 = (2 * pi) * R
            s0 = pltpu.async_copy(
                feat_hbm.at[idx_v.at[0, pl.ds(off0, R)]], sbuf.at[0], ssem)
            s1 = pltpu.async_copy(
                feat_hbm.at[idx_v.at[0, pl.ds(off0 + R, R)]], sbuf.at[1],
                ssem)
            prev = [
                pltpu.async_copy(
                    feat_hbm.at[chain_idx(q, off0, 0)], nbuf.at[q], csems[q])
                for q in range(4)
            ]
            for j in range(1, 5):
                for q in range(4):
                    prev[q].wait()
                nxt = [
                    pltpu.async_copy(
                        feat_hbm.at[chain_idx(q, off0, j)], nbuf.at[q],
                        csems[q], add=True)
                    for q in range(4)
                ]
                prev = nxt
            s0.wait()
            s1.wait()
            outs = [
                pltpu.async_copy(
                    sbuf.at[0], self_hbm.at[pl.ds(base + off0, R)], osem),
                pltpu.async_copy(
                    sbuf.at[1], self_hbm.at[pl.ds(base + off0 + R, R)], osem),
            ]
            for q in range(4):
                prev[q].wait()
                dst = nsa_hbm if q % 2 == 0 else nsb_hbm
                outs.append(pltpu.async_copy(
                    nbuf.at[q],
                    dst.at[pl.ds(base + off0 + (q // 2) * R, R)], osem))
            for cp in outs:
                cp.wait()
            return carry

        lax.fori_loop(0, nchunks // 2, pair, 0)

    return k(table, idxT)


def _tc_combine(self_rows, nsa, nsb, w1, w2, b):
    def body(x1, x2, x3, w1r, w2r, o):
        acc = jnp.dot(x1[...], w1r[...], preferred_element_type=jnp.float32)
        acc = acc + jnp.dot(x2[...] + x3[...], w2r[...],
                            preferred_element_type=jnp.float32) * \
            jnp.float32(0.1)
        o[...] = jnp.maximum(acc, 0.0)

    return pl.pallas_call(
        body,
        grid=((b + BLK - 1) // BLK,),
        in_specs=[
            pl.BlockSpec((BLK, D), lambda i: (i, 0)),
            pl.BlockSpec((BLK, D), lambda i: (i, 0)),
            pl.BlockSpec((BLK, D), lambda i: (i, 0)),
            pl.BlockSpec((D, D), lambda i: (0, 0)),
            pl.BlockSpec((D, D), lambda i: (0, 0)),
        ],
        out_specs=pl.BlockSpec((BLK, D), lambda i: (i, 0)),
        out_shape=jax.ShapeDtypeStruct((b, D), jnp.float32),
    )(self_rows, nsa, nsb, w1, w2)


def kernel(features, weight, nodes, neigh_idx):
    b = nodes.shape[0]
    step = NW * R * 2
    b_pad = ((b + step - 1) // step) * step

    idx_all = jnp.concatenate(
        [nodes[:, None].astype(jnp.int32), neigh_idx.astype(jnp.int32)],
        axis=1).T
    idxT = jnp.pad(idx_all, ((0, 0), (0, b_pad - b)))
    idxT = idxT.reshape(NSLOT, NW, b_pad // NW).transpose(1, 0, 2)

    self_rows, nsa, nsb = _sc_gather_sum(features, idxT, b_pad)
    return _tc_combine(self_rows, nsa, nsb, weight[:D], weight[D:], b)


# speedup vs baseline: 2.8737x; 1.1340x over previous
import functools

import jax
import jax.numpy as jnp
from jax import lax
from jax.experimental import pallas as pl
from jax.experimental.pallas import tpu as pltpu
from jax.experimental.pallas import tpu_sc as plsc

D = 128
NSLOT = 11
NC, NS = 2, 16
NW = NC * NS
R = 112
BLK = 2048


def _sc_gather_sum(table, idxT, b_pad):
    bpw = b_pad // NW
    nchunks = bpw // R
    dt = table.dtype
    mesh = plsc.VectorSubcoreMesh(core_axis_name="c", subcore_axis_name="s")

    @functools.partial(
        pl.kernel,
        out_type=(jax.ShapeDtypeStruct((b_pad, D), dt),
                  jax.ShapeDtypeStruct((b_pad, D), dt),
                  jax.ShapeDtypeStruct((b_pad, D), dt)),
        mesh=mesh,
        scratch_types=[
            pltpu.VMEM((NSLOT, bpw), jnp.int32),
            pltpu.VMEM((2, R, D), dt),
            pltpu.VMEM((4, R, D), dt),
            pltpu.SemaphoreType.DMA,
            pltpu.SemaphoreType.DMA,
            [pltpu.SemaphoreType.DMA] * 4,
        ],
        compiler_params=pltpu.CompilerParams(use_tc_tiling_on_sc=False),
    )
    def k(feat_hbm, idxT_hbm, self_hbm, nsa_hbm, nsb_hbm, idx_v, sbuf, nbuf,
          ssem, osem, csems):
        wid = lax.axis_index("s") * NC + lax.axis_index("c")
        base = wid * bpw
        pltpu.sync_copy(idxT_hbm.at[wid], idx_v)

        def chain_idx(q, off0, j):
            off = off0 + (q // 2) * R
            slot = 1 + (q % 2) * 5 + j
            return idx_v.at[slot, pl.ds(off, R)]

        def pair(pi, carry):
            off0 = (2 * pi) * R
            s0 = pltpu.async_copy(
                feat_hbm.at[idx_v.at[0, pl.ds(off0, R)]], sbuf.at[0], ssem)
            s1 = pltpu.async_copy(
                feat_hbm.at[idx_v.at[0, pl.ds(off0 + R, R)]], sbuf.at[1],
                ssem)
            prev = [
                pltpu.async_copy(
                    feat_hbm.at[chain_idx(q, off0, 0)], nbuf.at[q], csems[q])
                for q in range(4)
            ]
            for j in range(1, 5):
                for q in range(4):
                    prev[q].wait()
                nxt = [
                    pltpu.async_copy(
                        feat_hbm.at[chain_idx(q, off0, j)], nbuf.at[q],
                        csems[q], add=True)
                    for q in range(4)
                ]
                prev = nxt
            s0.wait()
            s1.wait()
            outs = [
                pltpu.async_copy(
                    sbuf.at[0], self_hbm.at[pl.ds(base + off0, R)], osem),
                pltpu.async_copy(
                    sbuf.at[1], self_hbm.at[pl.ds(base + off0 + R, R)], osem),
            ]
            for q in range(4):
                prev[q].wait()
                dst = nsa_hbm if q % 2 == 0 else nsb_hbm
                outs.append(pltpu.async_copy(
                    nbuf.at[q],
                    dst.at[pl.ds(base + off0 + (q // 2) * R, R)], osem))
            for cp in outs:
                cp.wait()
            return carry

        lax.fori_loop(0, nchunks // 2, pair, 0)

    return k(table, idxT)


def _tc_combine(self_rows, nsa, nsb, w1, w2, b):
    def body(x1, x2, x3, w1r, w2r, o):
        acc = jnp.dot(x1[...], w1r[...], preferred_element_type=jnp.float32)
        acc = acc + jnp.dot(x2[...] + x3[...], w2r[...],
                            preferred_element_type=jnp.float32) * \
            jnp.float32(0.1)
        o[...] = jnp.maximum(acc, 0.0)

    return pl.pallas_call(
        body,
        grid=((b + BLK - 1) // BLK,),
        in_specs=[
            pl.BlockSpec((BLK, D), lambda i: (i, 0)),
            pl.BlockSpec((BLK, D), lambda i: (i, 0)),
            pl.BlockSpec((BLK, D), lambda i: (i, 0)),
            pl.BlockSpec((D, D), lambda i: (0, 0)),
            pl.BlockSpec((D, D), lambda i: (0, 0)),
        ],
        out_specs=pl.BlockSpec((BLK, D), lambda i: (i, 0)),
        out_shape=jax.ShapeDtypeStruct((b, D), jnp.float32),
    )(self_rows, nsa, nsb, w1, w2)


def kernel(features, weight, nodes, neigh_idx):
    b = nodes.shape[0]
    step = NW * R * 2
    b_pad = ((b + step - 1) // step) * step

    idx_all = jnp.concatenate(
        [nodes[:, None].astype(jnp.int32), neigh_idx.astype(jnp.int32)],
        axis=1).T
    idxT = jnp.pad(idx_all, ((0, 0), (0, b_pad - b)))
    idxT = idxT.reshape(NSLOT, NW, b_pad // NW).transpose(1, 0, 2)

    self_rows, nsa, nsb = _sc_gather_sum(features, idxT, b_pad)
    return _tc_combine(self_rows, nsa, nsb, weight[:D], weight[D:], b)
